# Initial kernel scaffold; baseline (speedup 1.0000x reference)
import jax
import jax.numpy as jnp
from jax.experimental import pallas as pl


def kernel(x, rms_w, router_w, gate_up_w, down_w, residual_alpha):
    raise NotImplementedError("write your pallas kernel here")


# speedup vs baseline: 1.1854x; 1.1854x over previous
import functools

import jax
import jax.numpy as jnp
from jax.experimental import pallas as pl
from jax.experimental.pallas import tpu as pltpu

DIM = 768
HID = 2304
E = 8
EPS = 1e-06
BT = 256


def _moe_body(x_ref, rms_ref, router_ref, guw_ref, dw_ref, alpha_ref,
              out_ref):
    t = pl.program_id(0)
    e = pl.program_id(1)

    xb = x_ref[...]
    ms = jnp.mean(xb * xb, axis=1, keepdims=True)
    normed = xb * jax.lax.rsqrt(ms + EPS) * rms_ref[...]

    logits = jax.lax.dot_general(
        normed, router_ref[...], (((1,), (1,)), ((), ())),
        preferred_element_type=jnp.float32)
    m1 = jnp.max(logits, axis=1, keepdims=True)
    eq1 = logits == m1
    cols = []
    acc = jnp.zeros((BT, 1), jnp.float32)
    for i in range(E):
        cols.append(acc)
        acc = acc + eq1[:, i:i + 1].astype(jnp.float32)
    excl = jnp.concatenate(cols, axis=1)
    mask1 = eq1 & (excl == 0.0)
    l2 = jnp.where(mask1, -1e30, logits)
    m2 = jnp.max(l2, axis=1, keepdims=True)
    eq2 = l2 == m2
    cols = []
    acc = jnp.zeros((BT, 1), jnp.float32)
    for i in range(E):
        cols.append(acc)
        acc = acc + eq2[:, i:i + 1].astype(jnp.float32)
    excl2 = jnp.concatenate(cols, axis=1)
    mask2 = eq2 & (excl2 == 0.0)
    z = jnp.exp(m2 - m1)
    w1 = 1.0 / (1.0 + z)
    w2 = z / (1.0 + z)
    w_all = jnp.where(mask1, w1, 0.0) + jnp.where(mask2, w2, 0.0)
    eids = jax.lax.broadcasted_iota(jnp.int32, (BT, E), 1)
    w_e = jnp.sum(jnp.where(eids == e, w_all, 0.0), axis=1)

    nb = normed.astype(jnp.bfloat16)
    gu = jax.lax.dot_general(
        nb, guw_ref[0], (((1,), (1,)), ((), ())),
        preferred_element_type=jnp.float32)
    gate = gu[:, :HID]
    up = gu[:, HID:]
    hidden = (gate * jax.nn.sigmoid(gate) * up).astype(jnp.bfloat16)
    y = jax.lax.dot_general(
        hidden, dw_ref[0], (((1,), (1,)), ((), ())),
        preferred_element_type=jnp.float32)
    contrib = alpha_ref[0, 0] * w_e[:, None] * y

    @pl.when(e == 0)
    def _():
        out_ref[...] = xb + contrib

    @pl.when(e > 0)
    def _():
        out_ref[...] += contrib


def kernel(x, rms_w, router_w, gate_up_w, down_w, residual_alpha):
    B, S, D = x.shape
    T = B * S
    flat = x.reshape(T, D)
    alpha = jnp.clip(residual_alpha, -0.5, 2.0).reshape(1, 1)
    guw = gate_up_w.astype(jnp.bfloat16)
    dw = down_w.astype(jnp.bfloat16)
    nt = T // BT

    out = pl.pallas_call(
        _moe_body,
        grid=(nt, E),
        in_specs=[
            pl.BlockSpec((BT, D), lambda t, e: (t, 0)),
            pl.BlockSpec((1, D), lambda t, e: (0, 0)),
            pl.BlockSpec((E, D), lambda t, e: (0, 0)),
            pl.BlockSpec((1, 2 * HID, D), lambda t, e: (e, 0, 0)),
            pl.BlockSpec((1, D, HID), lambda t, e: (e, 0, 0)),
            pl.BlockSpec((1, 1), lambda t, e: (0, 0),
                         memory_space=pltpu.SMEM),
        ],
        out_specs=pl.BlockSpec((BT, D), lambda t, e: (t, 0)),
        out_shape=jax.ShapeDtypeStruct((T, D), jnp.float32),
        compiler_params=pltpu.CompilerParams(
            dimension_semantics=("arbitrary", "arbitrary")),
    )(flat, rms_w.reshape(1, D), router_w, guw, dw, alpha)
    return out.reshape(B, S, D)


# speedup vs baseline: 1.6363x; 1.3804x over previous
import functools

import jax
import jax.numpy as jnp
from jax import lax
from jax.experimental import pallas as pl
from jax.experimental.pallas import tpu as pltpu
from jax.experimental.pallas import tpu_sc as plsc

DIM = 768
HID = 2304
E = 8
EPS = 1e-06
TM = 512
NC, NS = 2, 16
NW = NC * NS


def _router_body(x_ref, rms_ref, rw_ref, normed_ref, pos_ref, wts_ref,
                 counts_ref):
    xb = x_ref[...]
    T = xb.shape[0]
    ms = jnp.mean(xb * xb, axis=1, keepdims=True)
    normed = xb * lax.rsqrt(ms + EPS) * rms_ref[...]
    normed_ref[...] = normed
    logits = lax.dot_general(normed, rw_ref[...], (((1,), (1,)), ((), ())),
                             preferred_element_type=jnp.float32)
    m1 = jnp.max(logits, axis=1, keepdims=True)
    eq1 = logits == m1
    cols, acc = [], jnp.zeros((T, 1), jnp.float32)
    for i in range(E):
        cols.append(acc)
        acc = acc + eq1[:, i:i + 1].astype(jnp.float32)
    mask1 = eq1 & (jnp.concatenate(cols, axis=1) == 0.0)
    l2 = jnp.where(mask1, -1e30, logits)
    m2 = jnp.max(l2, axis=1, keepdims=True)
    eq2 = l2 == m2
    cols, acc = [], jnp.zeros((T, 1), jnp.float32)
    for i in range(E):
        cols.append(acc)
        acc = acc + eq2[:, i:i + 1].astype(jnp.float32)
    mask2 = eq2 & (jnp.concatenate(cols, axis=1) == 0.0)
    z = jnp.exp(m2 - m1)
    w0 = 1.0 / (1.0 + z)
    w1 = z / (1.0 + z)
    wts_ref[...] = jnp.concatenate([w0, w1], axis=1)

    sel = mask1.astype(jnp.float32) + mask2.astype(jnp.float32)
    csum = sel
    sh = 1
    while sh < T:
        csum = csum + jnp.concatenate(
            [jnp.zeros((sh, E), jnp.float32), csum[:T - sh]], axis=0)
        sh *= 2
    cexcl = csum - sel
    counts = csum[T - 1:T, :]
    counts_ref[...] = counts.astype(jnp.int32)
    cols, acc = [], jnp.zeros((1, 1), jnp.float32)
    for i in range(E):
        cols.append(acc)
        acc = acc + counts[:, i:i + 1]
    offb = jnp.concatenate(cols, axis=1)
    posmat = offb + cexcl
    pos0 = jnp.sum(jnp.where(mask1, posmat, 0.0), axis=1)[None, :]
    pos1 = jnp.sum(jnp.where(mask2, posmat, 0.0), axis=1)[None, :]
    pos_ref[...] = jnp.concatenate([pos0, pos1], axis=0).astype(jnp.int32)


def _router_call(flat, rms_w, router_w):
    T, D = flat.shape
    return pl.pallas_call(
        _router_body,
        out_shape=(
            jax.ShapeDtypeStruct((T, D), jnp.float32),
            jax.ShapeDtypeStruct((2, T), jnp.int32),
            jax.ShapeDtypeStruct((T, 2), jnp.float32),
            jax.ShapeDtypeStruct((1, E), jnp.int32),
        ),
    )(flat, rms_w.reshape(1, D), router_w)


def _dispatch_call(normed, pos0, pos1):
    T, D = normed.shape
    CH = T // NW
    mesh = plsc.VectorSubcoreMesh(core_axis_name="c", subcore_axis_name="s")

    def body(normed_hbm, p0_hbm, p1_hbm, xs_hbm, i0_v, i1_v, rows_v, sem):
        wid = lax.axis_index("s") * NC + lax.axis_index("c")
        base = wid * CH
        pltpu.sync_copy(p0_hbm.at[pl.ds(base, CH)], i0_v)
        pltpu.sync_copy(p1_hbm.at[pl.ds(base, CH)], i1_v)
        pltpu.sync_copy(normed_hbm.at[pl.ds(base, CH)], rows_v)
        pltpu.async_copy(rows_v, xs_hbm.at[i0_v], sem).wait()
        pltpu.async_copy(rows_v, xs_hbm.at[i1_v], sem).wait()

    return pl.kernel(
        body,
        out_type=jax.ShapeDtypeStruct((2 * T, D), jnp.float32),
        mesh=mesh,
        scratch_types=[
            pltpu.VMEM((CH,), jnp.int32),
            pltpu.VMEM((CH,), jnp.int32),
            pltpu.VMEM((CH, D), jnp.float32),
            pltpu.SemaphoreType.DMA,
        ],
    )(normed, pos0, pos1)


def _combine_call(ys, pos0, pos1):
    M, D = ys.shape
    T = M // 2
    CH = T // NW
    mesh = plsc.VectorSubcoreMesh(core_axis_name="c", subcore_axis_name="s")

    def body(ys_hbm, p0_hbm, p1_hbm, g0_hbm, g1_hbm, i_v, rows_v, sem):
        wid = lax.axis_index("s") * NC + lax.axis_index("c")
        base = wid * CH
        pltpu.sync_copy(p0_hbm.at[pl.ds(base, CH)], i_v)
        pltpu.async_copy(ys_hbm.at[i_v], rows_v, sem).wait()
        pltpu.sync_copy(rows_v, g0_hbm.at[pl.ds(base, CH)])
        pltpu.sync_copy(p1_hbm.at[pl.ds(base, CH)], i_v)
        pltpu.async_copy(ys_hbm.at[i_v], rows_v, sem).wait()
        pltpu.sync_copy(rows_v, g1_hbm.at[pl.ds(base, CH)])

    return pl.kernel(
        body,
        out_type=(
            jax.ShapeDtypeStruct((T, D), jnp.float32),
            jax.ShapeDtypeStruct((T, D), jnp.float32),
        ),
        mesh=mesh,
        scratch_types=[
            pltpu.VMEM((CH,), jnp.int32),
            pltpu.VMEM((CH, D), jnp.float32),
            pltpu.SemaphoreType.DMA,
        ],
    )(ys, pos0, pos1)


def _schedule(counts, M):
    ntiles = M // TM
    nstep = ntiles + E - 1
    offs = jnp.concatenate(
        [jnp.zeros((1,), jnp.int32), jnp.cumsum(counts, dtype=jnp.int32)])
    lo = jnp.minimum(offs[:E] // TM, ntiles - 1)
    hi_ne = jnp.maximum(offs[1:] - 1, 0) // TM
    hi = jnp.maximum(jnp.where(counts > 0, hi_ne, lo), lo)
    span = hi - lo + 1
    sg = jnp.concatenate(
        [jnp.zeros((1,), jnp.int32), jnp.cumsum(span, dtype=jnp.int32)])
    i_arr = jnp.arange(nstep, dtype=jnp.int32)
    gidx = jnp.clip(
        jnp.sum((sg[None, :E] <= i_arr[:, None]).astype(jnp.int32), axis=1)
        - 1, 0, E - 1)
    j_arr = jnp.clip(lo[gidx] + (i_arr - sg[gidx]), 0, ntiles - 1)
    valid = i_arr < sg[E]
    rs = jnp.where(valid, jnp.maximum(offs[gidx], j_arr * TM), 0)
    re = jnp.where(valid, jnp.minimum(offs[gidx + 1], (j_arr + 1) * TM), 0)
    fv = jnp.concatenate(
        [jnp.ones((1,), jnp.int32),
         (j_arr[1:] != j_arr[:-1]).astype(jnp.int32)])
    return jnp.stack([j_arr, gidx, rs, re, fv])


def _gmm_body(s_ref, xs_ref, guw_ref, dw_ref, ys_ref):
    i = pl.program_id(0)
    j = s_ref[0, i]
    rs = s_ref[2, i]
    re = s_ref[3, i]
    fv = s_ref[4, i]
    lhs = xs_ref[...].astype(jnp.bfloat16)
    gu = lax.dot_general(lhs, guw_ref[0], (((1,), (1,)), ((), ())),
                         preferred_element_type=jnp.float32)
    gate = gu[:, :HID]
    up = gu[:, HID:]
    hidden = (gate * jax.nn.sigmoid(gate) * up).astype(jnp.bfloat16)
    y = lax.dot_general(hidden, dw_ref[0], (((1,), (1,)), ((), ())),
                        preferred_element_type=jnp.float32)
    rows = j * TM + lax.broadcasted_iota(jnp.int32, (TM, 1), 0)
    contrib = jnp.where((rows >= rs) & (rows < re), y, 0.0)

    @pl.when(fv == 1)
    def _():
        ys_ref[...] = contrib

    @pl.when(fv == 0)
    def _():
        ys_ref[...] += contrib


def _gmm_call(xs, guw, dw, sched):
    M, D = xs.shape
    nstep = sched.shape[1]
    grid_spec = pltpu.PrefetchScalarGridSpec(
        num_scalar_prefetch=1,
        grid=(nstep,),
        in_specs=[
            pl.BlockSpec((TM, D), lambda i, s: (s[0, i], 0)),
            pl.BlockSpec((1, 2 * HID, D), lambda i, s: (s[1, i], 0, 0)),
            pl.BlockSpec((1, D, HID), lambda i, s: (s[1, i], 0, 0)),
        ],
        out_specs=pl.BlockSpec((TM, D), lambda i, s: (s[0, i], 0)),
    )
    return pl.pallas_call(
        _gmm_body,
        grid_spec=grid_spec,
        out_shape=jax.ShapeDtypeStruct((M, D), jnp.float32),
        compiler_params=pltpu.CompilerParams(
            dimension_semantics=("arbitrary",)),
    )(sched, xs, guw, dw)


def _final_body(x_ref, g0_ref, g1_ref, w_ref, alpha_ref, out_ref):
    w = w_ref[...]
    moe = w[:, 0:1] * g0_ref[...] + w[:, 1:2] * g1_ref[...]
    out_ref[...] = x_ref[...] + alpha_ref[0, 0] * moe


def _final_call(flat, g0, g1, wts, alpha):
    T, D = flat.shape
    BTF = 512
    return pl.pallas_call(
        _final_body,
        grid=(T // BTF,),
        in_specs=[
            pl.BlockSpec((BTF, D), lambda t: (t, 0)),
            pl.BlockSpec((BTF, D), lambda t: (t, 0)),
            pl.BlockSpec((BTF, D), lambda t: (t, 0)),
            pl.BlockSpec((BTF, 2), lambda t: (t, 0)),
            pl.BlockSpec((1, 1), lambda t: (0, 0), memory_space=pltpu.SMEM),
        ],
        out_specs=pl.BlockSpec((BTF, D), lambda t: (t, 0)),
        out_shape=jax.ShapeDtypeStruct((T, D), jnp.float32),
    )(flat, g0, g1, wts, alpha)


def kernel(x, rms_w, router_w, gate_up_w, down_w, residual_alpha):
    B, S, D = x.shape
    T = B * S
    M = 2 * T
    flat = x.reshape(T, D)
    alpha = jnp.clip(residual_alpha, -0.5, 2.0).reshape(1, 1)
    guw = gate_up_w.astype(jnp.bfloat16)
    dw = down_w.astype(jnp.bfloat16)

    normed, pos, wts, counts = _router_call(flat, rms_w, router_w)
    pos0 = pos[0]
    pos1 = pos[1]
    sched = _schedule(counts[0], M)
    xs = _dispatch_call(normed, pos0, pos1)
    ys = _gmm_call(xs, guw, dw, sched)
    g0, g1 = _combine_call(ys, pos0, pos1)
    out = _final_call(flat, g0, g1, wts, alpha)
    return out.reshape(B, S, D)


# speedup vs baseline: 2.0813x; 1.2720x over previous
import functools

import jax
import jax.numpy as jnp
from jax import lax
from jax.experimental import pallas as pl
from jax.experimental.pallas import tpu as pltpu
from jax.experimental.pallas import tpu_sc as plsc

DIM = 768
HID = 2304
E = 8
EPS = 1e-06
TM = 512
NC, NS = 2, 16
NW = NC * NS


def _router_body(x_ref, rms_ref, rw_ref, normed_ref, pos_ref, wts_ref,
                 counts_ref):
    xb = x_ref[...]
    T = xb.shape[0]
    ms = jnp.mean(xb * xb, axis=1, keepdims=True)
    normed = xb * lax.rsqrt(ms + EPS) * rms_ref[...]
    normed_ref[...] = normed
    logits = lax.dot_general(normed, rw_ref[...], (((1,), (1,)), ((), ())),
                             preferred_element_type=jnp.float32)
    m1 = jnp.max(logits, axis=1, keepdims=True)
    eq1 = logits == m1
    cols, acc = [], jnp.zeros((T, 1), jnp.float32)
    for i in range(E):
        cols.append(acc)
        acc = acc + eq1[:, i:i + 1].astype(jnp.float32)
    mask1 = eq1 & (jnp.concatenate(cols, axis=1) == 0.0)
    l2 = jnp.where(mask1, -1e30, logits)
    m2 = jnp.max(l2, axis=1, keepdims=True)
    eq2 = l2 == m2
    cols, acc = [], jnp.zeros((T, 1), jnp.float32)
    for i in range(E):
        cols.append(acc)
        acc = acc + eq2[:, i:i + 1].astype(jnp.float32)
    mask2 = eq2 & (jnp.concatenate(cols, axis=1) == 0.0)
    z = jnp.exp(m2 - m1)
    w0 = 1.0 / (1.0 + z)
    w1 = z / (1.0 + z)
    wts_ref[...] = jnp.concatenate([w0, w1], axis=1)

    sel = mask1.astype(jnp.float32) + mask2.astype(jnp.float32)
    csum = sel
    sh = 1
    while sh < T:
        csum = csum + jnp.concatenate(
            [jnp.zeros((sh, E), jnp.float32), csum[:T - sh]], axis=0)
        sh *= 2
    cexcl = csum - sel
    counts = csum[T - 1:T, :]
    counts_ref[...] = counts.astype(jnp.int32)
    cols, acc = [], jnp.zeros((1, 1), jnp.float32)
    for i in range(E):
        cols.append(acc)
        acc = acc + counts[:, i:i + 1]
    offb = jnp.concatenate(cols, axis=1)
    posmat = offb + cexcl
    pos0 = jnp.sum(jnp.where(mask1, posmat, 0.0), axis=1)[None, :]
    pos1 = jnp.sum(jnp.where(mask2, posmat, 0.0), axis=1)[None, :]
    pos_ref[...] = jnp.concatenate([pos0, pos1], axis=0).astype(jnp.int32)


def _router_call(flat, rms_w, router_w):
    T, D = flat.shape
    return pl.pallas_call(
        _router_body,
        out_shape=(
            jax.ShapeDtypeStruct((T, D), jnp.float32),
            jax.ShapeDtypeStruct((2, T), jnp.int32),
            jax.ShapeDtypeStruct((T, 2), jnp.float32),
            jax.ShapeDtypeStruct((1, E), jnp.int32),
        ),
    )(flat, rms_w.reshape(1, D), router_w)


def _dispatch_call(normed, pos0, pos1):
    T, D = normed.shape
    CH = T // NW
    mesh = plsc.VectorSubcoreMesh(core_axis_name="c", subcore_axis_name="s")

    def body(normed_hbm, p0_hbm, p1_hbm, xs_hbm, i0_v, i1_v, rows_v, sem):
        wid = lax.axis_index("s") * NC + lax.axis_index("c")
        base = wid * CH
        pltpu.sync_copy(p0_hbm.at[pl.ds(base, CH)], i0_v)
        pltpu.sync_copy(p1_hbm.at[pl.ds(base, CH)], i1_v)
        pltpu.sync_copy(normed_hbm.at[pl.ds(base, CH)], rows_v)
        pltpu.async_copy(rows_v, xs_hbm.at[i0_v], sem).wait()
        pltpu.async_copy(rows_v, xs_hbm.at[i1_v], sem).wait()

    return pl.kernel(
        body,
        out_type=jax.ShapeDtypeStruct((2 * T, D), jnp.float32),
        mesh=mesh,
        scratch_types=[
            pltpu.VMEM((CH,), jnp.int32),
            pltpu.VMEM((CH,), jnp.int32),
            pltpu.VMEM((CH, D), jnp.float32),
            pltpu.SemaphoreType.DMA,
        ],
    )(normed, pos0, pos1)


def _combine_call(ys, pos0, pos1):
    M, D = ys.shape
    T = M // 2
    CH = T // NW
    mesh = plsc.VectorSubcoreMesh(core_axis_name="c", subcore_axis_name="s")

    def body(ys_hbm, p0_hbm, p1_hbm, g0_hbm, g1_hbm, i_v, rows_v, sem):
        wid = lax.axis_index("s") * NC + lax.axis_index("c")
        base = wid * CH
        pltpu.sync_copy(p0_hbm.at[pl.ds(base, CH)], i_v)
        pltpu.async_copy(ys_hbm.at[i_v], rows_v, sem).wait()
        pltpu.sync_copy(rows_v, g0_hbm.at[pl.ds(base, CH)])
        pltpu.sync_copy(p1_hbm.at[pl.ds(base, CH)], i_v)
        pltpu.async_copy(ys_hbm.at[i_v], rows_v, sem).wait()
        pltpu.sync_copy(rows_v, g1_hbm.at[pl.ds(base, CH)])

    return pl.kernel(
        body,
        out_type=(
            jax.ShapeDtypeStruct((T, D), jnp.float32),
            jax.ShapeDtypeStruct((T, D), jnp.float32),
        ),
        mesh=mesh,
        scratch_types=[
            pltpu.VMEM((CH,), jnp.int32),
            pltpu.VMEM((CH, D), jnp.float32),
            pltpu.SemaphoreType.DMA,
        ],
    )(ys, pos0, pos1)


def _schedule(counts, M):
    ntiles = M // TM
    nstep = ntiles + E - 1
    offs = jnp.concatenate(
        [jnp.zeros((1,), jnp.int32), jnp.cumsum(counts, dtype=jnp.int32)])
    lo = jnp.minimum(offs[:E] // TM, ntiles - 1)
    hi_ne = jnp.maximum(offs[1:] - 1, 0) // TM
    hi = jnp.maximum(jnp.where(counts > 0, hi_ne, lo), lo)
    span = hi - lo + 1
    sg = jnp.concatenate(
        [jnp.zeros((1,), jnp.int32), jnp.cumsum(span, dtype=jnp.int32)])
    i_arr = jnp.arange(nstep, dtype=jnp.int32)
    gidx = jnp.clip(
        jnp.sum((sg[None, :E] <= i_arr[:, None]).astype(jnp.int32), axis=1)
        - 1, 0, E - 1)
    j_arr = jnp.clip(lo[gidx] + (i_arr - sg[gidx]), 0, ntiles - 1)
    valid = i_arr < sg[E]
    rs = jnp.where(valid, jnp.maximum(offs[gidx], j_arr * TM), 0)
    re = jnp.where(valid, jnp.minimum(offs[gidx + 1], (j_arr + 1) * TM), 0)
    fv = jnp.concatenate(
        [jnp.ones((1,), jnp.int32),
         (j_arr[1:] != j_arr[:-1]).astype(jnp.int32)])
    return jnp.stack([j_arr, gidx, rs, re, fv])


HC = HID // 2


def _gmm_body(s_ref, xs_ref, gw_ref, uw_ref, dw_ref, ys_ref):
    i = pl.program_id(0)
    c = pl.program_id(1)
    j = s_ref[0, i]
    rs = s_ref[2, i]
    re = s_ref[3, i]
    fv = s_ref[4, i]
    lhs = xs_ref[...].astype(jnp.bfloat16)
    gate = lax.dot_general(lhs, gw_ref[0, 0].astype(jnp.bfloat16),
                           (((1,), (1,)), ((), ())),
                           preferred_element_type=jnp.float32)
    up = lax.dot_general(lhs, uw_ref[0, 0].astype(jnp.bfloat16),
                         (((1,), (1,)), ((), ())),
                         preferred_element_type=jnp.float32)
    hidden = (gate * jax.nn.sigmoid(gate) * up).astype(jnp.bfloat16)
    y = lax.dot_general(hidden, dw_ref[0].astype(jnp.bfloat16),
                        (((1,), (1,)), ((), ())),
                        preferred_element_type=jnp.float32)
    rows = j * TM + lax.broadcasted_iota(jnp.int32, (TM, 1), 0)
    contrib = jnp.where((rows >= rs) & (rows < re), y, 0.0)

    @pl.when((fv == 1) & (c == 0))
    def _():
        ys_ref[...] = contrib

    @pl.when((fv == 0) | (c != 0))
    def _():
        ys_ref[...] += contrib


def _gmm_call(xs, gate_up_w, down_w, sched):
    M, D = xs.shape
    nstep = sched.shape[1]
    guw4 = gate_up_w.reshape(E, 4, HC, D)
    grid_spec = pltpu.PrefetchScalarGridSpec(
        num_scalar_prefetch=1,
        grid=(nstep, 2),
        in_specs=[
            pl.BlockSpec((TM, D), lambda i, c, s: (s[0, i], 0)),
            pl.BlockSpec((1, 1, HC, D), lambda i, c, s: (s[1, i], c, 0, 0)),
            pl.BlockSpec((1, 1, HC, D),
                         lambda i, c, s: (s[1, i], c + 2, 0, 0)),
            pl.BlockSpec((1, D, HC), lambda i, c, s: (s[1, i], 0, c)),
        ],
        out_specs=pl.BlockSpec((TM, D), lambda i, c, s: (s[0, i], 0)),
    )
    return pl.pallas_call(
        _gmm_body,
        grid_spec=grid_spec,
        out_shape=jax.ShapeDtypeStruct((M, D), jnp.float32),
        compiler_params=pltpu.CompilerParams(
            dimension_semantics=("arbitrary", "arbitrary")),
    )(sched, xs, guw4, guw4, down_w)


def _final_body(x_ref, g0_ref, g1_ref, w_ref, alpha_ref, out_ref):
    w = w_ref[...]
    moe = w[:, 0:1] * g0_ref[...] + w[:, 1:2] * g1_ref[...]
    out_ref[...] = x_ref[...] + alpha_ref[0, 0] * moe


def _final_call(flat, g0, g1, wts, alpha):
    T, D = flat.shape
    BTF = 512
    return pl.pallas_call(
        _final_body,
        grid=(T // BTF,),
        in_specs=[
            pl.BlockSpec((BTF, D), lambda t: (t, 0)),
            pl.BlockSpec((BTF, D), lambda t: (t, 0)),
            pl.BlockSpec((BTF, D), lambda t: (t, 0)),
            pl.BlockSpec((BTF, 2), lambda t: (t, 0)),
            pl.BlockSpec((1, 1), lambda t: (0, 0), memory_space=pltpu.SMEM),
        ],
        out_specs=pl.BlockSpec((BTF, D), lambda t: (t, 0)),
        out_shape=jax.ShapeDtypeStruct((T, D), jnp.float32),
    )(flat, g0, g1, wts, alpha)


def kernel(x, rms_w, router_w, gate_up_w, down_w, residual_alpha):
    B, S, D = x.shape
    T = B * S
    M = 2 * T
    flat = x.reshape(T, D)
    alpha = jnp.clip(residual_alpha, -0.5, 2.0).reshape(1, 1)

    normed, pos, wts, counts = _router_call(flat, rms_w, router_w)
    pos0 = pos[0]
    pos1 = pos[1]
    sched = _schedule(counts[0], M)
    xs = _dispatch_call(normed, pos0, pos1)
    ys = _gmm_call(xs, gate_up_w, down_w, sched)
    g0, g1 = _combine_call(ys, pos0, pos1)
    out = _final_call(flat, g0, g1, wts, alpha)
    return out.reshape(B, S, D)
